# Initial kernel scaffold; baseline (speedup 1.0000x reference)
import jax
import jax.numpy as jnp
from jax.experimental import pallas as pl


def kernel(x, edge_index, W1, att_src1, att_dst1, b1, W2, att_src2, att_dst2, b2, Wa, ba, Wb, bb):
    raise NotImplementedError("write your pallas kernel here")


# speedup vs baseline: 1.1573x; 1.1573x over previous
import functools
import jax
import jax.numpy as jnp
from jax.experimental import pallas as pl

_N = 100000
_E = 1600000
_BN = 2000


def _d1_body(x_ref, w1_ref, as1_ref, ad1_ref, h_ref, asn_ref, adn_ref):
    x = x_ref[...].astype(jnp.bfloat16)
    w1 = w1_ref[...].astype(jnp.bfloat16)
    h = jnp.dot(x, w1, preferred_element_type=jnp.float32)
    h_ref[...] = h
    hh = h.reshape(_BN, 2, 16)
    asn_ref[...] = jnp.sum(hh * as1_ref[...][None], axis=-1)
    adn_ref[...] = jnp.sum(hh * ad1_ref[...][None], axis=-1)


def _dense1(x, W1, att_src1, att_dst1):
    grid = (_N // _BN,)
    return pl.pallas_call(
        _d1_body,
        grid=grid,
        in_specs=[
            pl.BlockSpec((_BN, 3), lambda i: (i, 0)),
            pl.BlockSpec((3, 32), lambda i: (0, 0)),
            pl.BlockSpec((2, 16), lambda i: (0, 0)),
            pl.BlockSpec((2, 16), lambda i: (0, 0)),
        ],
        out_specs=[
            pl.BlockSpec((_BN, 32), lambda i: (i, 0)),
            pl.BlockSpec((_BN, 2), lambda i: (i, 0)),
            pl.BlockSpec((_BN, 2), lambda i: (i, 0)),
        ],
        out_shape=[
            jax.ShapeDtypeStruct((_N, 32), jnp.float32),
            jax.ShapeDtypeStruct((_N, 2), jnp.float32),
            jax.ShapeDtypeStruct((_N, 2), jnp.float32),
        ],
    )(x, W1, att_src1, att_dst1)


def _edge_layer(h, asn, adn, src, dst, heads, ch):
    logit = asn[src] + adn[dst]
    w = jnp.exp(jnp.where(logit >= 0, logit, 0.2 * logit))
    hs = h.reshape(_N, heads, ch)[src]
    num = jax.ops.segment_sum(hs * w[:, :, None], dst, num_segments=_N)
    den = jax.ops.segment_sum(w, dst, num_segments=_N)
    return num / (den[:, :, None] + 1e-16)


def kernel(x, edge_index, W1, att_src1, att_dst1, b1, W2, att_src2, att_dst2,
           b2, Wa, ba, Wb, bb):
    src = edge_index[0]
    dst = edge_index[1]
    h1, as1, ad1 = _dense1(x, W1, att_src1, att_dst1)
    o1 = _edge_layer(h1, as1, ad1, src, dst, 2, 16)
    h1p = jax.nn.elu(o1.reshape(_N, 32) + b1)
    h2t = jnp.dot(h1p.astype(jnp.bfloat16), W2.astype(jnp.bfloat16),
                  preferred_element_type=jnp.float32)
    as2 = jnp.sum(h2t * att_src2, axis=-1, keepdims=True)
    ad2 = jnp.sum(h2t * att_dst2, axis=-1, keepdims=True)
    o2 = _edge_layer(h2t, as2, ad2, src, dst, 1, 16)
    h2 = o2.reshape(_N, 16) + b2
    ef = jnp.concatenate([h2[src], h2[dst]], axis=-1)
    hmid = jax.nn.relu(jnp.dot(ef.astype(jnp.bfloat16),
                               Wa.astype(jnp.bfloat16),
                               preferred_element_type=jnp.float32) + ba)
    flows = (jnp.dot(hmid.astype(jnp.bfloat16), Wb.astype(jnp.bfloat16),
                     preferred_element_type=jnp.float32) + bb).squeeze(-1)
    return jax.nn.relu(flows)


# speedup vs baseline: 1.1820x; 1.0213x over previous
import functools
import jax
import jax.numpy as jnp
from jax import lax
from jax.experimental import pallas as pl
from jax.experimental.pallas import tpu as pltpu
from jax.experimental.pallas import tpu_sc as plsc

_N = 100000
_E = 1600000
_BN = 2000
_BE = 8000
_R = _E // 128
_SS = 4


def _d1_body(x_ref, w1_ref, as1_ref, ad1_ref, h_ref, asn_ref, adn_ref):
    x = x_ref[...].astype(jnp.bfloat16)
    w1 = w1_ref[...].astype(jnp.bfloat16)
    h = jnp.dot(x, w1, preferred_element_type=jnp.float32)
    h_ref[...] = h
    hh = h.reshape(_BN, 2, 16)
    asn_ref[...] = jnp.sum(hh * as1_ref[...][None], axis=-1)
    adn_ref[...] = jnp.sum(hh * ad1_ref[...][None], axis=-1)


def _dense1(x, W1, att_src1, att_dst1):
    return pl.pallas_call(
        _d1_body,
        grid=(_N // _BN,),
        in_specs=[
            pl.BlockSpec((_BN, 3), lambda i: (i, 0)),
            pl.BlockSpec((3, 32), lambda i: (0, 0)),
            pl.BlockSpec((2, 16), lambda i: (0, 0)),
            pl.BlockSpec((2, 16), lambda i: (0, 0)),
        ],
        out_specs=[
            pl.BlockSpec((_BN, 32), lambda i: (i, 0)),
            pl.BlockSpec((_BN, 2), lambda i: (i, 0)),
            pl.BlockSpec((_BN, 2), lambda i: (i, 0)),
        ],
        out_shape=[
            jax.ShapeDtypeStruct((_N, 32), jnp.float32),
            jax.ShapeDtypeStruct((_N, 2), jnp.float32),
            jax.ShapeDtypeStruct((_N, 2), jnp.float32),
        ],
    )(x, W1, att_src1, att_dst1)


def _p3_gather(h2f, srcb, dstb):
    mesh = plsc.VectorSubcoreMesh(core_axis_name="c", subcore_axis_name="s")

    @functools.partial(
        pl.kernel,
        out_type=jax.ShapeDtypeStruct((2, _E, 16), jnp.float32),
        mesh=mesh,
        compiler_params=pltpu.CompilerParams(use_tc_tiling_on_sc=False),
        scratch_types=[
            pltpu.VMEM((_SS, 1, 128), jnp.int32),
            pltpu.VMEM((_SS, 1, 128), jnp.int32),
            pltpu.VMEM((_SS * 128, 16), jnp.float32),
            pltpu.VMEM((_SS * 128, 16), jnp.float32),
            pltpu.SemaphoreType.DMA,
        ],
    )
    def k(h2f_hbm, srcb_hbm, dstb_hbm, ef_hbm, sv, dv, srows, drows, sem):
        cid = lax.axis_index("c")
        sid = lax.axis_index("s")
        w = sid * 2 + cid
        r0 = (_R * w) // 32
        r1 = (_R * (w + 1)) // 32
        nss = (r1 - r0) // _SS

        @pl.loop(0, nss)
        def _(t):
            rb = r0 + t * _SS
            pltpu.sync_copy(srcb_hbm.at[pl.ds(rb, _SS)], sv)
            pltpu.sync_copy(dstb_hbm.at[pl.ds(rb, _SS)], dv)
            hs = []
            for j in range(_SS):
                hs.append(pltpu.async_copy(
                    h2f_hbm.at[sv.at[j, 0]], srows.at[pl.ds(j * 128, 128)], sem))
                hs.append(pltpu.async_copy(
                    h2f_hbm.at[dv.at[j, 0]], drows.at[pl.ds(j * 128, 128)], sem))
            for h in hs:
                h.wait()
            pltpu.sync_copy(srows, ef_hbm.at[0].at[pl.ds(rb * 128, _SS * 128)])
            pltpu.sync_copy(drows, ef_hbm.at[1].at[pl.ds(rb * 128, _SS * 128)])

        @pl.loop(r0 + nss * _SS, r1)
        def _(r):
            pltpu.sync_copy(srcb_hbm.at[pl.ds(r, 1)], sv.at[pl.ds(0, 1)])
            pltpu.sync_copy(dstb_hbm.at[pl.ds(r, 1)], dv.at[pl.ds(0, 1)])
            pltpu.async_copy(
                h2f_hbm.at[sv.at[0, 0]], srows.at[pl.ds(0, 128)], sem).wait()
            pltpu.async_copy(
                h2f_hbm.at[dv.at[0, 0]], drows.at[pl.ds(0, 128)], sem).wait()
            pltpu.sync_copy(srows.at[pl.ds(0, 128)],
                            ef_hbm.at[0].at[pl.ds(r * 128, 128)])
            pltpu.sync_copy(drows.at[pl.ds(0, 128)],
                            ef_hbm.at[1].at[pl.ds(r * 128, 128)])

    return k(h2f, srcb, dstb)


def _d4_body(ef0_ref, ef1_ref, wa_ref, ba_ref, wb_ref, bb_ref, out_ref):
    e0 = ef0_ref[...][0]
    e1 = ef1_ref[...][0]
    efc = jnp.concatenate([e0, e1], axis=1).astype(jnp.bfloat16)
    hmid = jnp.dot(efc, wa_ref[...].astype(jnp.bfloat16),
                   preferred_element_type=jnp.float32) + ba_ref[...]
    hmid = jnp.maximum(hmid, 0.0)
    fl = jnp.dot(hmid.astype(jnp.bfloat16), wb_ref[...].astype(jnp.bfloat16),
                 preferred_element_type=jnp.float32) + bb_ref[...]
    out_ref[...] = jnp.maximum(fl, 0.0)


def _dense4(ef, Wa, ba, Wb, bb):
    return pl.pallas_call(
        _d4_body,
        grid=(_E // _BE,),
        in_specs=[
            pl.BlockSpec((1, _BE, 16), lambda i: (0, i, 0)),
            pl.BlockSpec((1, _BE, 16), lambda i: (1, i, 0)),
            pl.BlockSpec((32, 16), lambda i: (0, 0)),
            pl.BlockSpec((1, 16), lambda i: (0, 0)),
            pl.BlockSpec((16, 1), lambda i: (0, 0)),
            pl.BlockSpec((1, 1), lambda i: (0, 0)),
        ],
        out_specs=pl.BlockSpec((_BE, 1), lambda i: (i, 0)),
        out_shape=jax.ShapeDtypeStruct((_E, 1), jnp.float32),
    )(ef, ef, Wa, ba.reshape(1, 16), Wb, bb.reshape(1, 1))


def _edge_layer(h, asn, adn, src, dst, heads, ch):
    logit = asn[src] + adn[dst]
    w = jnp.exp(jnp.where(logit >= 0, logit, 0.2 * logit))
    hs = h.reshape(_N, heads, ch)[src]
    num = jax.ops.segment_sum(hs * w[:, :, None], dst, num_segments=_N)
    den = jax.ops.segment_sum(w, dst, num_segments=_N)
    return num / (den[:, :, None] + 1e-16)


def kernel(x, edge_index, W1, att_src1, att_dst1, b1, W2, att_src2, att_dst2,
           b2, Wa, ba, Wb, bb):
    src = edge_index[0]
    dst = edge_index[1]
    srcb = src.reshape(_R, 1, 128)
    dstb = dst.reshape(_R, 1, 128)
    h1, as1, ad1 = _dense1(x, W1, att_src1, att_dst1)
    o1 = _edge_layer(h1, as1, ad1, src, dst, 2, 16)
    h1p = jax.nn.elu(o1.reshape(_N, 32) + b1)
    h2t = jnp.dot(h1p.astype(jnp.bfloat16), W2.astype(jnp.bfloat16),
                  preferred_element_type=jnp.float32)
    as2 = jnp.sum(h2t * att_src2, axis=-1, keepdims=True)
    ad2 = jnp.sum(h2t * att_dst2, axis=-1, keepdims=True)
    o2 = _edge_layer(h2t, as2, ad2, src, dst, 1, 16)
    h2 = o2.reshape(_N, 16) + b2
    ef = _p3_gather(h2, srcb, dstb)
    flows = _dense4(ef, Wa, ba, Wb, bb)
    return flows.reshape(_E)


# speedup vs baseline: 36.8517x; 31.1780x over previous
import functools
import jax
import jax.numpy as jnp
from jax import lax
from jax.experimental import pallas as pl
from jax.experimental.pallas import tpu as pltpu
from jax.experimental.pallas import tpu_sc as plsc

_N = 100000
_E = 1600000
_BN = 2000
_BE = 8000
_R = _E // 128
_SS = 4
_SSE = 2
_NSUB = _N // 16

_SC_PARAMS = pltpu.CompilerParams(use_tc_tiling_on_sc=False,
                                  needs_layout_passes=False)


def _iota16():
    return lax.iota(jnp.int32, 16)


def _full16(v):
    return jnp.full((16,), v, jnp.int32)


def _d1_body(x_ref, w1_ref, as1_ref, ad1_ref, t1_ref, aux_ref):
    x = x_ref[...].astype(jnp.bfloat16)
    w1h = w1_ref[...][0].astype(jnp.bfloat16)
    h = jnp.dot(x, w1h, preferred_element_type=jnp.float32)
    as_col = jnp.sum(h * as1_ref[...][0], axis=1, keepdims=True)
    ad_col = jnp.sum(h * ad1_ref[...][0], axis=1, keepdims=True)
    t1_ref[...] = h[None]
    aux_ref[...] = jnp.concatenate(
        [as_col, ad_col, jnp.zeros((_BN, 6), jnp.float32)], axis=1)[None]


def _dense1(x, W1, att_src1, att_dst1):
    w1t = W1.reshape(3, 2, 16).transpose(1, 0, 2)
    return pl.pallas_call(
        _d1_body,
        grid=(2, _N // _BN),
        in_specs=[
            pl.BlockSpec((_BN, 3), lambda h, i: (i, 0)),
            pl.BlockSpec((1, 3, 16), lambda h, i: (h, 0, 0)),
            pl.BlockSpec((1, 1, 16), lambda h, i: (h, 0, 0)),
            pl.BlockSpec((1, 1, 16), lambda h, i: (h, 0, 0)),
        ],
        out_specs=[
            pl.BlockSpec((1, _BN, 16), lambda h, i: (h, i, 0)),
            pl.BlockSpec((1, _BN, 8), lambda h, i: (h, i, 0)),
        ],
        out_shape=[
            jax.ShapeDtypeStruct((2, _N, 16), jnp.float32),
            jax.ShapeDtypeStruct((2, _N, 8), jnp.float32),
        ],
    )(x, w1t, att_src1.reshape(2, 1, 16), att_dst1.reshape(2, 1, 16))


def _edge_pass(Tg, aux, srcb, dstb, per_core_head):
    mesh = plsc.VectorSubcoreMesh(core_axis_name="c", subcore_axis_name="s")
    _NA = _N + _N // 16

    @functools.partial(
        pl.kernel,
        out_type=jax.ShapeDtypeStruct((2 * _NA, 16), jnp.float32),
        mesh=mesh,
        compiler_params=_SC_PARAMS,
        scratch_types=[
            pltpu.VMEM_SHARED((_NA, 16), jnp.float32),
            pltpu.VMEM((_SSE * 128, 8), jnp.float32),
            pltpu.VMEM((_SSE * 128, 8), jnp.float32),
            pltpu.VMEM((_SSE, 1, 128), jnp.int32),
            pltpu.VMEM((_SSE, 1, 128), jnp.int32),
            pltpu.VMEM((_SSE, 1, 128), jnp.int32),
            pltpu.VMEM((_SSE, 1, 128), jnp.int32),
            pltpu.VMEM((_SSE * 128, 16), jnp.float32),
            pltpu.VMEM((_SSE * 128, 16), jnp.float32),
            pltpu.VMEM((_SSE * 128, 16), jnp.float32),
            pltpu.SemaphoreType.DMA,
        ],
    )
    def k(t_hbm, aux_hbm, srcb_hbm, dstb_hbm, u_hbm,
          acc, auxs, auxd, sv, dv, dvo, dvw, rows, outb, outw, sem):
        cid = lax.axis_index("c")
        sid = lax.axis_index("s")
        z16 = jnp.zeros((16,), jnp.float32)
        for r in range(_SSE * 128):
            outw[r, pl.ds(0, 16)] = z16
            outb[r, pl.ds(0, 16)] = z16
        nz = _SSE * 128
        a0 = (_NA * sid) // 16
        a1 = (_NA * (sid + 1)) // 16
        nfull = (a1 - a0) // nz

        @pl.loop(0, nfull)
        def _(p):
            pltpu.sync_copy(outb, acc.at[pl.ds(a0 + p * nz, nz)])

        @pl.loop(a0 + nfull * nz, a1)
        def _(p):
            pltpu.sync_copy(outb.at[pl.ds(0, 1)], acc.at[pl.ds(p, 1)])
        plsc.subcore_barrier()

        off16 = jnp.full((16,), cid * _N, jnp.int32)

        def prep_ids(nj):
            for j in range(nj):
                for g2 in range(8):
                    sl = pl.ds(g2 * 16, 16)
                    d_raw = dv[j, 0, sl]
                    dvw[j, 0, sl] = _full16(_N) + lax.shift_right_logical(
                        d_raw, 4)
                    if per_core_head:
                        sv[j, 0, sl] = sv[j, 0, sl] + off16
                        dvo[j, 0, sl] = d_raw + off16
                    else:
                        dvo[j, 0, sl] = d_raw

        if per_core_head:
            w_id, nw = sid, 16
        else:
            w_id, nw = sid * 2 + cid, 32
        r0 = (_R * w_id) // nw
        r1 = (_R * (w_id + 1)) // nw
        nss = (r1 - r0) // _SSE

        def compute16(base, dst16):
            idx16 = _full16(base) + _iota16()
            as16 = plsc.load_gather(auxs, [idx16, _full16(0)])
            ad16 = plsc.load_gather(auxd, [idx16, _full16(1)])
            logit = as16 + ad16
            w16 = jnp.exp(jnp.where(logit >= 0, logit, 0.2 * logit))
            md16 = lax.bitwise_and(dst16, _full16(15))
            plsc.store_scatter(outw, [idx16, md16], w16)
            for kk in range(16):
                ck = plsc.load_gather(rows, [idx16, _full16(kk)])
                plsc.store_scatter(outb, [idx16, _full16(kk)], ck * w16)

        def do_block(nj):
            prep_ids(nj)
            hs = [pltpu.async_copy(t_hbm.at[sv.at[j, 0]],
                                   rows.at[pl.ds(j * 128, 128)], sem)
                  for j in range(nj)]
            hs += [pltpu.async_copy(aux_hbm.at[sv.at[j, 0]],
                                    auxs.at[pl.ds(j * 128, 128)], sem)
                   for j in range(nj)]
            hs += [pltpu.async_copy(aux_hbm.at[dvo.at[j, 0]],
                                    auxd.at[pl.ds(j * 128, 128)], sem)
                   for j in range(nj)]
            for h in hs:
                h.wait()
            for j in range(nj):
                for g in range(8):
                    compute16(j * 128 + g * 16, dv[j, 0, pl.ds(g * 16, 16)])
            for j in range(nj):
                pltpu.sync_copy(outb.at[pl.ds(j * 128, 128)],
                                acc.at[dv.at[j, 0]], add=True)
                pltpu.sync_copy(outw.at[pl.ds(j * 128, 128)],
                                acc.at[dvw.at[j, 0]], add=True)
            for j in range(nj):
                for g in range(8):
                    base = j * 128 + g * 16
                    idx16 = _full16(base) + _iota16()
                    dst16 = dv[j, 0, pl.ds(g * 16, 16)]
                    md16 = lax.bitwise_and(dst16, _full16(15))
                    plsc.store_scatter(outw, [idx16, md16], z16)

        @pl.loop(0, nss)
        def _(t):
            rb = r0 + t * _SSE
            pltpu.sync_copy(srcb_hbm.at[pl.ds(rb, _SSE)], sv)
            pltpu.sync_copy(dstb_hbm.at[pl.ds(rb, _SSE)], dv)
            do_block(_SSE)

        @pl.loop(r0 + nss * _SSE, r1)
        def _(r):
            pltpu.sync_copy(srcb_hbm.at[pl.ds(r, 1)], sv.at[pl.ds(0, 1)])
            pltpu.sync_copy(dstb_hbm.at[pl.ds(r, 1)], dv.at[pl.ds(0, 1)])
            do_block(1)

        plsc.subcore_barrier()
        pltpu.sync_copy(acc.at[pl.ds(a0, a1 - a0)],
                        u_hbm.at[pl.ds(cid * _NA + a0, a1 - a0)])

    return k(Tg, aux, srcb, dstb)


def _d2_body(uh_ref, den_ref, b1_ref, w2_ref, as2_ref, ad2_ref,
             t2_ref, aux_ref):
    uh = uh_ref[...]
    den = den_ref[...]
    o0 = uh[0] / (den[0] + 1e-16)
    o1 = uh[1] / (den[1] + 1e-16)
    cat = jnp.concatenate([o0, o1], axis=1) + b1_ref[...]
    h1p = jnp.where(cat > 0, cat, jnp.exp(cat) - 1.0)
    h2t = jnp.dot(h1p.astype(jnp.bfloat16), w2_ref[...].astype(jnp.bfloat16),
                  preferred_element_type=jnp.float32)
    as_col = jnp.sum(h2t * as2_ref[...], axis=1, keepdims=True)
    ad_col = jnp.sum(h2t * ad2_ref[...], axis=1, keepdims=True)
    t2_ref[...] = h2t
    aux_ref[...] = jnp.concatenate(
        [as_col, ad_col, jnp.zeros((_BN, 6), jnp.float32)], axis=1)


def _dense2(U1h, den1, b1, W2, att_src2, att_dst2):
    return pl.pallas_call(
        _d2_body,
        grid=(_N // _BN,),
        in_specs=[
            pl.BlockSpec((2, _BN, 16), lambda i: (0, i, 0)),
            pl.BlockSpec((2, _BN, 1), lambda i: (0, i, 0)),
            pl.BlockSpec((1, 32), lambda i: (0, 0)),
            pl.BlockSpec((32, 16), lambda i: (0, 0)),
            pl.BlockSpec((1, 16), lambda i: (0, 0)),
            pl.BlockSpec((1, 16), lambda i: (0, 0)),
        ],
        out_specs=[
            pl.BlockSpec((_BN, 16), lambda i: (i, 0)),
            pl.BlockSpec((_BN, 8), lambda i: (i, 0)),
        ],
        out_shape=[
            jax.ShapeDtypeStruct((_N, 16), jnp.float32),
            jax.ShapeDtypeStruct((_N, 8), jnp.float32),
        ],
    )(U1h, den1, b1.reshape(1, 32), W2, att_src2, att_dst2)


def _d3_body(uh_ref, den_ref, b2_ref, out_ref):
    u = uh_ref[...][0] + uh_ref[...][1]
    dd = den_ref[...][0] + den_ref[...][1]
    out_ref[...] = u / (dd + 1e-16) + b2_ref[...]


def _dense3(U2h, den2, b2):
    return pl.pallas_call(
        _d3_body,
        grid=(_N // _BN,),
        in_specs=[
            pl.BlockSpec((2, _BN, 16), lambda i: (0, i, 0)),
            pl.BlockSpec((2, _BN, 1), lambda i: (0, i, 0)),
            pl.BlockSpec((1, 16), lambda i: (0, 0)),
        ],
        out_specs=pl.BlockSpec((_BN, 16), lambda i: (i, 0)),
        out_shape=jax.ShapeDtypeStruct((_N, 16), jnp.float32),
    )(U2h, den2, b2.reshape(1, 16))


def _p3_gather(h2f, srcb, dstb):
    mesh = plsc.VectorSubcoreMesh(core_axis_name="c", subcore_axis_name="s")

    @functools.partial(
        pl.kernel,
        out_type=jax.ShapeDtypeStruct((2, _E, 16), jnp.float32),
        mesh=mesh,
        compiler_params=_SC_PARAMS,
        scratch_types=[
            pltpu.VMEM((_SS, 1, 128), jnp.int32),
            pltpu.VMEM((_SS, 1, 128), jnp.int32),
            pltpu.VMEM((_SS * 128, 16), jnp.float32),
            pltpu.VMEM((_SS * 128, 16), jnp.float32),
            pltpu.SemaphoreType.DMA,
        ],
    )
    def k(h2f_hbm, srcb_hbm, dstb_hbm, ef_hbm, sv, dv, srows, drows, sem):
        cid = lax.axis_index("c")
        sid = lax.axis_index("s")
        w = sid * 2 + cid
        r0 = (_R * w) // 32
        r1 = (_R * (w + 1)) // 32
        nss = (r1 - r0) // _SS

        @pl.loop(0, nss)
        def _(t):
            rb = r0 + t * _SS
            pltpu.sync_copy(srcb_hbm.at[pl.ds(rb, _SS)], sv)
            pltpu.sync_copy(dstb_hbm.at[pl.ds(rb, _SS)], dv)
            hs = []
            for j in range(_SS):
                hs.append(pltpu.async_copy(
                    h2f_hbm.at[sv.at[j, 0]], srows.at[pl.ds(j * 128, 128)], sem))
                hs.append(pltpu.async_copy(
                    h2f_hbm.at[dv.at[j, 0]], drows.at[pl.ds(j * 128, 128)], sem))
            for h in hs:
                h.wait()
            pltpu.sync_copy(srows, ef_hbm.at[0].at[pl.ds(rb * 128, _SS * 128)])
            pltpu.sync_copy(drows, ef_hbm.at[1].at[pl.ds(rb * 128, _SS * 128)])

        @pl.loop(r0 + nss * _SS, r1)
        def _(r):
            pltpu.sync_copy(srcb_hbm.at[pl.ds(r, 1)], sv.at[pl.ds(0, 1)])
            pltpu.sync_copy(dstb_hbm.at[pl.ds(r, 1)], dv.at[pl.ds(0, 1)])
            pltpu.async_copy(
                h2f_hbm.at[sv.at[0, 0]], srows.at[pl.ds(0, 128)], sem).wait()
            pltpu.async_copy(
                h2f_hbm.at[dv.at[0, 0]], drows.at[pl.ds(0, 128)], sem).wait()
            pltpu.sync_copy(srows.at[pl.ds(0, 128)],
                            ef_hbm.at[0].at[pl.ds(r * 128, 128)])
            pltpu.sync_copy(drows.at[pl.ds(0, 128)],
                            ef_hbm.at[1].at[pl.ds(r * 128, 128)])

    return k(h2f, srcb, dstb)


def _d4_body(ef0_ref, ef1_ref, wa_ref, ba_ref, wb_ref, bb_ref, out_ref):
    e0 = ef0_ref[...][0]
    e1 = ef1_ref[...][0]
    efc = jnp.concatenate([e0, e1], axis=1).astype(jnp.bfloat16)
    hmid = jnp.dot(efc, wa_ref[...].astype(jnp.bfloat16),
                   preferred_element_type=jnp.float32) + ba_ref[...]
    hmid = jnp.maximum(hmid, 0.0)
    fl = jnp.dot(hmid.astype(jnp.bfloat16), wb_ref[...].astype(jnp.bfloat16),
                 preferred_element_type=jnp.float32) + bb_ref[...]
    out_ref[...] = jnp.maximum(fl, 0.0)


def _dense4(ef, Wa, ba, Wb, bb):
    return pl.pallas_call(
        _d4_body,
        grid=(_E // _BE,),
        in_specs=[
            pl.BlockSpec((1, _BE, 16), lambda i: (0, i, 0)),
            pl.BlockSpec((1, _BE, 16), lambda i: (1, i, 0)),
            pl.BlockSpec((32, 16), lambda i: (0, 0)),
            pl.BlockSpec((1, 16), lambda i: (0, 0)),
            pl.BlockSpec((16, 1), lambda i: (0, 0)),
            pl.BlockSpec((1, 1), lambda i: (0, 0)),
        ],
        out_specs=pl.BlockSpec((_BE, 1), lambda i: (i, 0)),
        out_shape=jax.ShapeDtypeStruct((_E, 1), jnp.float32),
    )(ef, ef, Wa, ba.reshape(1, 16), Wb, bb.reshape(1, 1))


def kernel(x, edge_index, W1, att_src1, att_dst1, b1, W2, att_src2, att_dst2,
           b2, Wa, ba, Wb, bb):
    srcb = edge_index[0].reshape(_R, 1, 128)
    dstb = edge_index[1].reshape(_R, 1, 128)
    na = _N + _N // 16
    Tg1, aux1 = _dense1(x, W1, att_src1, att_dst1)
    U1 = _edge_pass(Tg1.reshape(2 * _N, 16), aux1.reshape(2 * _N, 8),
                    srcb, dstb, True).reshape(2, na, 16)
    U1h = U1[:, :_N, :]
    den1 = U1[:, _N:, :].reshape(2, _N, 1)
    T2, aux2 = _dense2(U1h, den1, b1, W2, att_src2, att_dst2)
    U2 = _edge_pass(T2, aux2, srcb, dstb, False).reshape(2, na, 16)
    U2h = U2[:, :_N, :]
    den2 = U2[:, _N:, :].reshape(2, _N, 1)
    h2f = _dense3(U2h, den2, b2)
    ef = _p3_gather(h2f, srcb, dstb)
    flows = _dense4(ef, Wa, ba, Wb, bb)
    return flows.reshape(_E)


# speedup vs baseline: 37.5045x; 1.0177x over previous
import functools
import jax
import jax.numpy as jnp
from jax import lax
from jax.experimental import pallas as pl
from jax.experimental.pallas import tpu as pltpu
from jax.experimental.pallas import tpu_sc as plsc

_N = 100000
_E = 1600000
_BN = 2000
_BE = 8000
_R = _E // 128
_SS = 4
_SSE = 2
_NSUB = _N // 16

_SC_PARAMS = pltpu.CompilerParams(use_tc_tiling_on_sc=False,
                                  needs_layout_passes=False)


def _iota16():
    return lax.iota(jnp.int32, 16)


def _full16(v):
    return jnp.full((16,), v, jnp.int32)


def _d1_body(x_ref, w1_ref, as1_ref, ad1_ref, t1_ref, aux_ref):
    x = x_ref[...].astype(jnp.bfloat16)
    w1h = w1_ref[...][0].astype(jnp.bfloat16)
    h = jnp.dot(x, w1h, preferred_element_type=jnp.float32)
    as_col = jnp.sum(h * as1_ref[...][0], axis=1, keepdims=True)
    ad_col = jnp.sum(h * ad1_ref[...][0], axis=1, keepdims=True)
    t1_ref[...] = h[None]
    aux_ref[...] = jnp.concatenate(
        [as_col, ad_col, jnp.zeros((_BN, 6), jnp.float32)], axis=1)[None]


def _dense1(x, W1, att_src1, att_dst1):
    w1t = W1.reshape(3, 2, 16).transpose(1, 0, 2)
    return pl.pallas_call(
        _d1_body,
        grid=(2, _N // _BN),
        in_specs=[
            pl.BlockSpec((_BN, 3), lambda h, i: (i, 0)),
            pl.BlockSpec((1, 3, 16), lambda h, i: (h, 0, 0)),
            pl.BlockSpec((1, 1, 16), lambda h, i: (h, 0, 0)),
            pl.BlockSpec((1, 1, 16), lambda h, i: (h, 0, 0)),
        ],
        out_specs=[
            pl.BlockSpec((1, _BN, 16), lambda h, i: (h, i, 0)),
            pl.BlockSpec((1, _BN, 8), lambda h, i: (h, i, 0)),
        ],
        out_shape=[
            jax.ShapeDtypeStruct((2, _N, 16), jnp.float32),
            jax.ShapeDtypeStruct((2, _N, 8), jnp.float32),
        ],
    )(x, w1t, att_src1.reshape(2, 1, 16), att_dst1.reshape(2, 1, 16))


def _edge_pass(Tg, aux, srcb, dstb, per_core_head):
    mesh = plsc.VectorSubcoreMesh(core_axis_name="c", subcore_axis_name="s")
    _NA = _N + _N // 16

    @functools.partial(
        pl.kernel,
        out_type=jax.ShapeDtypeStruct((2 * _NA, 16), jnp.float32),
        mesh=mesh,
        compiler_params=_SC_PARAMS,
        scratch_types=[
            pltpu.VMEM_SHARED((_NA, 16), jnp.float32),
            pltpu.VMEM((_SSE * 128, 8), jnp.float32),
            pltpu.VMEM((_SSE * 128, 8), jnp.float32),
            pltpu.VMEM((_SSE, 1, 128), jnp.int32),
            pltpu.VMEM((_SSE, 1, 128), jnp.int32),
            pltpu.VMEM((_SSE, 1, 128), jnp.int32),
            pltpu.VMEM((_SSE, 1, 128), jnp.int32),
            pltpu.VMEM((_SSE * 128, 16), jnp.float32),
            pltpu.VMEM((_SSE * 128, 16), jnp.float32),
            pltpu.VMEM((_SSE * 128, 16), jnp.float32),
            pltpu.SemaphoreType.DMA,
        ],
    )
    def k(t_hbm, aux_hbm, srcb_hbm, dstb_hbm, u_hbm,
          acc, auxs, auxd, sv, dv, dvo, dvw, rows, outb, outw, sem):
        cid = lax.axis_index("c")
        sid = lax.axis_index("s")
        z16 = jnp.zeros((16,), jnp.float32)
        for r in range(_SSE * 128):
            outw[r, pl.ds(0, 16)] = z16
            outb[r, pl.ds(0, 16)] = z16
        nz = _SSE * 128
        a0 = (_NA * sid) // 16
        a1 = (_NA * (sid + 1)) // 16
        nfull = (a1 - a0) // nz

        @pl.loop(0, nfull)
        def _(p):
            pltpu.sync_copy(outb, acc.at[pl.ds(a0 + p * nz, nz)])

        @pl.loop(a0 + nfull * nz, a1)
        def _(p):
            pltpu.sync_copy(outb.at[pl.ds(0, 1)], acc.at[pl.ds(p, 1)])
        plsc.subcore_barrier()

        off16 = jnp.full((16,), cid * _N, jnp.int32)

        def prep_ids(nj):
            for j in range(nj):
                for g2 in range(8):
                    sl = pl.ds(g2 * 16, 16)
                    d_raw = dv[j, 0, sl]
                    dvw[j, 0, sl] = _full16(_N) + lax.shift_right_logical(
                        d_raw, 4)
                    if per_core_head:
                        sv[j, 0, sl] = sv[j, 0, sl] + off16
                        dvo[j, 0, sl] = d_raw + off16
                    else:
                        dvo[j, 0, sl] = d_raw

        if per_core_head:
            w_id, nw = sid, 16
        else:
            w_id, nw = sid * 2 + cid, 32
        r0 = (_R * w_id) // nw
        r1 = (_R * (w_id + 1)) // nw
        nss = (r1 - r0) // _SSE

        def compute16(base, dst16):
            idx16 = _full16(base) + _iota16()
            as16 = plsc.load_gather(auxs, [idx16, _full16(0)])
            ad16 = plsc.load_gather(auxd, [idx16, _full16(1)])
            logit = as16 + ad16
            w16 = jnp.exp(jnp.where(logit >= 0, logit, 0.2 * logit))
            md16 = lax.bitwise_and(dst16, _full16(15))
            plsc.store_scatter(outw, [idx16, md16], w16)
            for kk in range(16):
                ck = plsc.load_gather(rows, [idx16, _full16(kk)])
                plsc.store_scatter(outb, [idx16, _full16(kk)], ck * w16)

        def do_block(nj):
            prep_ids(nj)
            hs = [pltpu.async_copy(t_hbm.at[sv.at[j, 0]],
                                   rows.at[pl.ds(j * 128, 128)], sem)
                  for j in range(nj)]
            hs += [pltpu.async_copy(aux_hbm.at[sv.at[j, 0]],
                                    auxs.at[pl.ds(j * 128, 128)], sem)
                   for j in range(nj)]
            hs += [pltpu.async_copy(aux_hbm.at[dvo.at[j, 0]],
                                    auxd.at[pl.ds(j * 128, 128)], sem)
                   for j in range(nj)]
            for h in hs:
                h.wait()
            for j in range(nj):
                for g in range(8):
                    compute16(j * 128 + g * 16, dv[j, 0, pl.ds(g * 16, 16)])
            hs = [pltpu.async_copy(outb.at[pl.ds(j * 128, 128)],
                                   acc.at[dv.at[j, 0]], sem, add=True)
                  for j in range(nj)]
            hs += [pltpu.async_copy(outw.at[pl.ds(j * 128, 128)],
                                    acc.at[dvw.at[j, 0]], sem, add=True)
                   for j in range(nj)]
            for h in hs:
                h.wait()
            for j in range(nj):
                for g in range(8):
                    base = j * 128 + g * 16
                    idx16 = _full16(base) + _iota16()
                    dst16 = dv[j, 0, pl.ds(g * 16, 16)]
                    md16 = lax.bitwise_and(dst16, _full16(15))
                    plsc.store_scatter(outw, [idx16, md16], z16)

        @pl.loop(0, nss)
        def _(t):
            rb = r0 + t * _SSE
            pltpu.sync_copy(srcb_hbm.at[pl.ds(rb, _SSE)], sv)
            pltpu.sync_copy(dstb_hbm.at[pl.ds(rb, _SSE)], dv)
            do_block(_SSE)

        @pl.loop(r0 + nss * _SSE, r1)
        def _(r):
            pltpu.sync_copy(srcb_hbm.at[pl.ds(r, 1)], sv.at[pl.ds(0, 1)])
            pltpu.sync_copy(dstb_hbm.at[pl.ds(r, 1)], dv.at[pl.ds(0, 1)])
            do_block(1)

        plsc.subcore_barrier()
        pltpu.sync_copy(acc.at[pl.ds(a0, a1 - a0)],
                        u_hbm.at[pl.ds(cid * _NA + a0, a1 - a0)])

    return k(Tg, aux, srcb, dstb)


def _d2_body(uh_ref, den_ref, b1_ref, w2_ref, as2_ref, ad2_ref,
             t2_ref, aux_ref):
    uh = uh_ref[...]
    den = den_ref[...]
    o0 = uh[0] / (den[0] + 1e-16)
    o1 = uh[1] / (den[1] + 1e-16)
    cat = jnp.concatenate([o0, o1], axis=1) + b1_ref[...]
    h1p = jnp.where(cat > 0, cat, jnp.exp(cat) - 1.0)
    h2t = jnp.dot(h1p.astype(jnp.bfloat16), w2_ref[...].astype(jnp.bfloat16),
                  preferred_element_type=jnp.float32)
    as_col = jnp.sum(h2t * as2_ref[...], axis=1, keepdims=True)
    ad_col = jnp.sum(h2t * ad2_ref[...], axis=1, keepdims=True)
    t2_ref[...] = h2t
    aux_ref[...] = jnp.concatenate(
        [as_col, ad_col, jnp.zeros((_BN, 6), jnp.float32)], axis=1)


def _dense2(U1h, den1, b1, W2, att_src2, att_dst2):
    return pl.pallas_call(
        _d2_body,
        grid=(_N // _BN,),
        in_specs=[
            pl.BlockSpec((2, _BN, 16), lambda i: (0, i, 0)),
            pl.BlockSpec((2, _BN, 1), lambda i: (0, i, 0)),
            pl.BlockSpec((1, 32), lambda i: (0, 0)),
            pl.BlockSpec((32, 16), lambda i: (0, 0)),
            pl.BlockSpec((1, 16), lambda i: (0, 0)),
            pl.BlockSpec((1, 16), lambda i: (0, 0)),
        ],
        out_specs=[
            pl.BlockSpec((_BN, 16), lambda i: (i, 0)),
            pl.BlockSpec((_BN, 8), lambda i: (i, 0)),
        ],
        out_shape=[
            jax.ShapeDtypeStruct((_N, 16), jnp.float32),
            jax.ShapeDtypeStruct((_N, 8), jnp.float32),
        ],
    )(U1h, den1, b1.reshape(1, 32), W2, att_src2, att_dst2)


def _d3_body(uh_ref, den_ref, b2_ref, out_ref):
    u = uh_ref[...][0] + uh_ref[...][1]
    dd = den_ref[...][0] + den_ref[...][1]
    out_ref[...] = u / (dd + 1e-16) + b2_ref[...]


def _dense3(U2h, den2, b2):
    return pl.pallas_call(
        _d3_body,
        grid=(_N // _BN,),
        in_specs=[
            pl.BlockSpec((2, _BN, 16), lambda i: (0, i, 0)),
            pl.BlockSpec((2, _BN, 1), lambda i: (0, i, 0)),
            pl.BlockSpec((1, 16), lambda i: (0, 0)),
        ],
        out_specs=pl.BlockSpec((_BN, 16), lambda i: (i, 0)),
        out_shape=jax.ShapeDtypeStruct((_N, 16), jnp.float32),
    )(U2h, den2, b2.reshape(1, 16))


def _p3_gather(h2f, srcb, dstb):
    mesh = plsc.VectorSubcoreMesh(core_axis_name="c", subcore_axis_name="s")

    @functools.partial(
        pl.kernel,
        out_type=jax.ShapeDtypeStruct((2, _E, 16), jnp.float32),
        mesh=mesh,
        compiler_params=_SC_PARAMS,
        scratch_types=[
            pltpu.VMEM((_SS, 1, 128), jnp.int32),
            pltpu.VMEM((_SS, 1, 128), jnp.int32),
            pltpu.VMEM((_SS * 128, 16), jnp.float32),
            pltpu.VMEM((_SS * 128, 16), jnp.float32),
            pltpu.SemaphoreType.DMA,
        ],
    )
    def k(h2f_hbm, srcb_hbm, dstb_hbm, ef_hbm, sv, dv, srows, drows, sem):
        cid = lax.axis_index("c")
        sid = lax.axis_index("s")
        w = sid * 2 + cid
        r0 = (_R * w) // 32
        r1 = (_R * (w + 1)) // 32
        nss = (r1 - r0) // _SS

        @pl.loop(0, nss)
        def _(t):
            rb = r0 + t * _SS
            pltpu.sync_copy(srcb_hbm.at[pl.ds(rb, _SS)], sv)
            pltpu.sync_copy(dstb_hbm.at[pl.ds(rb, _SS)], dv)
            hs = []
            for j in range(_SS):
                hs.append(pltpu.async_copy(
                    h2f_hbm.at[sv.at[j, 0]], srows.at[pl.ds(j * 128, 128)], sem))
                hs.append(pltpu.async_copy(
                    h2f_hbm.at[dv.at[j, 0]], drows.at[pl.ds(j * 128, 128)], sem))
            for h in hs:
                h.wait()
            pltpu.sync_copy(srows, ef_hbm.at[0].at[pl.ds(rb * 128, _SS * 128)])
            pltpu.sync_copy(drows, ef_hbm.at[1].at[pl.ds(rb * 128, _SS * 128)])

        @pl.loop(r0 + nss * _SS, r1)
        def _(r):
            pltpu.sync_copy(srcb_hbm.at[pl.ds(r, 1)], sv.at[pl.ds(0, 1)])
            pltpu.sync_copy(dstb_hbm.at[pl.ds(r, 1)], dv.at[pl.ds(0, 1)])
            pltpu.async_copy(
                h2f_hbm.at[sv.at[0, 0]], srows.at[pl.ds(0, 128)], sem).wait()
            pltpu.async_copy(
                h2f_hbm.at[dv.at[0, 0]], drows.at[pl.ds(0, 128)], sem).wait()
            pltpu.sync_copy(srows.at[pl.ds(0, 128)],
                            ef_hbm.at[0].at[pl.ds(r * 128, 128)])
            pltpu.sync_copy(drows.at[pl.ds(0, 128)],
                            ef_hbm.at[1].at[pl.ds(r * 128, 128)])

    return k(h2f, srcb, dstb)


def _d4_body(ef0_ref, ef1_ref, wa_ref, ba_ref, wb_ref, bb_ref, out_ref):
    e0 = ef0_ref[...][0]
    e1 = ef1_ref[...][0]
    efc = jnp.concatenate([e0, e1], axis=1).astype(jnp.bfloat16)
    hmid = jnp.dot(efc, wa_ref[...].astype(jnp.bfloat16),
                   preferred_element_type=jnp.float32) + ba_ref[...]
    hmid = jnp.maximum(hmid, 0.0)
    fl = jnp.dot(hmid.astype(jnp.bfloat16), wb_ref[...].astype(jnp.bfloat16),
                 preferred_element_type=jnp.float32) + bb_ref[...]
    out_ref[...] = jnp.maximum(fl, 0.0)


def _dense4(ef, Wa, ba, Wb, bb):
    return pl.pallas_call(
        _d4_body,
        grid=(_E // _BE,),
        in_specs=[
            pl.BlockSpec((1, _BE, 16), lambda i: (0, i, 0)),
            pl.BlockSpec((1, _BE, 16), lambda i: (1, i, 0)),
            pl.BlockSpec((32, 16), lambda i: (0, 0)),
            pl.BlockSpec((1, 16), lambda i: (0, 0)),
            pl.BlockSpec((16, 1), lambda i: (0, 0)),
            pl.BlockSpec((1, 1), lambda i: (0, 0)),
        ],
        out_specs=pl.BlockSpec((_BE, 1), lambda i: (i, 0)),
        out_shape=jax.ShapeDtypeStruct((_E, 1), jnp.float32),
    )(ef, ef, Wa, ba.reshape(1, 16), Wb, bb.reshape(1, 1))


def kernel(x, edge_index, W1, att_src1, att_dst1, b1, W2, att_src2, att_dst2,
           b2, Wa, ba, Wb, bb):
    srcb = edge_index[0].reshape(_R, 1, 128)
    dstb = edge_index[1].reshape(_R, 1, 128)
    na = _N + _N // 16
    Tg1, aux1 = _dense1(x, W1, att_src1, att_dst1)
    U1 = _edge_pass(Tg1.reshape(2 * _N, 16), aux1.reshape(2 * _N, 8),
                    srcb, dstb, True).reshape(2, na, 16)
    U1h = U1[:, :_N, :]
    den1 = U1[:, _N:, :].reshape(2, _N, 1)
    T2, aux2 = _dense2(U1h, den1, b1, W2, att_src2, att_dst2)
    U2 = _edge_pass(T2, aux2, srcb, dstb, False).reshape(2, na, 16)
    U2h = U2[:, :_N, :]
    den2 = U2[:, _N:, :].reshape(2, _N, 1)
    h2f = _dense3(U2h, den2, b2)
    ef = _p3_gather(h2f, srcb, dstb)
    flows = _dense4(ef, Wa, ba, Wb, bb)
    return flows.reshape(_E)


# speedup vs baseline: 48.8814x; 1.3033x over previous
import functools
import jax
import jax.numpy as jnp
from jax import lax
from jax.experimental import pallas as pl
from jax.experimental.pallas import tpu as pltpu
from jax.experimental.pallas import tpu_sc as plsc

_N = 100000
_E = 1600000
_BN = 2000
_BE = 8000
_R = _E // 128
_SS = 4
_SSE = 2
_NSUB = _N // 16

_SC_PARAMS = pltpu.CompilerParams(use_tc_tiling_on_sc=False,
                                  needs_layout_passes=False)


def _iota16():
    return lax.iota(jnp.int32, 16)


def _full16(v):
    return jnp.full((16,), v, jnp.int32)


def _d1_body(x_ref, w1_ref, as1_ref, ad1_ref, t1_ref, aux_ref):
    x = x_ref[...].astype(jnp.bfloat16)
    w1h = w1_ref[...][0].astype(jnp.bfloat16)
    h = jnp.dot(x, w1h, preferred_element_type=jnp.float32)
    as_col = jnp.sum(h * as1_ref[...][0], axis=1, keepdims=True)
    ad_col = jnp.sum(h * ad1_ref[...][0], axis=1, keepdims=True)
    t1_ref[...] = h[None]
    aux_ref[...] = jnp.concatenate(
        [as_col, ad_col, jnp.zeros((_BN, 6), jnp.float32)], axis=1)[None]


def _dense1(x, W1, att_src1, att_dst1):
    w1t = W1.reshape(3, 2, 16).transpose(1, 0, 2)
    return pl.pallas_call(
        _d1_body,
        grid=(2, _N // _BN),
        in_specs=[
            pl.BlockSpec((_BN, 3), lambda h, i: (i, 0)),
            pl.BlockSpec((1, 3, 16), lambda h, i: (h, 0, 0)),
            pl.BlockSpec((1, 1, 16), lambda h, i: (h, 0, 0)),
            pl.BlockSpec((1, 1, 16), lambda h, i: (h, 0, 0)),
        ],
        out_specs=[
            pl.BlockSpec((1, _BN, 16), lambda h, i: (h, i, 0)),
            pl.BlockSpec((1, _BN, 8), lambda h, i: (h, i, 0)),
        ],
        out_shape=[
            jax.ShapeDtypeStruct((2, _N, 16), jnp.float32),
            jax.ShapeDtypeStruct((2, _N, 8), jnp.float32),
        ],
    )(x, w1t, att_src1.reshape(2, 1, 16), att_dst1.reshape(2, 1, 16))


def _edge_pass(Tg, aux, srcb, dstb, per_core_head):
    mesh = plsc.VectorSubcoreMesh(core_axis_name="c", subcore_axis_name="s")
    _NA = _N + _N // 16

    @functools.partial(
        pl.kernel,
        out_type=jax.ShapeDtypeStruct((2 * _NA, 16), jnp.float32),
        mesh=mesh,
        compiler_params=_SC_PARAMS,
        scratch_types=[
            pltpu.VMEM_SHARED((_NA, 16), jnp.float32),
            pltpu.VMEM((_SSE * 128, 8), jnp.float32),
            pltpu.VMEM((_SSE * 128, 8), jnp.float32),
            pltpu.VMEM((_SSE, 1, 128), jnp.int32),
            pltpu.VMEM((_SSE, 1, 128), jnp.int32),
            pltpu.VMEM((_SSE, 1, 128), jnp.int32),
            pltpu.VMEM((_SSE, 1, 128), jnp.int32),
            pltpu.VMEM((_SSE * 128, 16), jnp.float32),
            pltpu.VMEM((_SSE * 128, 16), jnp.float32),
            pltpu.VMEM((_SSE * 128, 16), jnp.float32),
            pltpu.SemaphoreType.DMA,
        ],
    )
    def k(t_hbm, aux_hbm, srcb_hbm, dstb_hbm, u_hbm,
          acc, auxs, auxd, sv, dv, dvo, dvw, rows, outb, outw, sem):
        cid = lax.axis_index("c")
        sid = lax.axis_index("s")
        z16 = jnp.zeros((16,), jnp.float32)
        for r in range(_SSE * 128):
            outw[r, pl.ds(0, 16)] = z16
            outb[r, pl.ds(0, 16)] = z16
        nz = _SSE * 128
        a0 = (_NA * sid) // 16
        a1 = (_NA * (sid + 1)) // 16
        nfull = (a1 - a0) // nz

        @pl.loop(0, nfull)
        def _(p):
            pltpu.sync_copy(outb, acc.at[pl.ds(a0 + p * nz, nz)])

        @pl.loop(a0 + nfull * nz, a1)
        def _(p):
            pltpu.sync_copy(outb.at[pl.ds(0, 1)], acc.at[pl.ds(p, 1)])
        plsc.subcore_barrier()

        off16 = jnp.full((16,), cid * _N, jnp.int32)

        def prep_ids(nj):
            for j in range(nj):
                for g2 in range(8):
                    sl = pl.ds(g2 * 16, 16)
                    d_raw = dv[j, 0, sl]
                    dvw[j, 0, sl] = _full16(_N) + lax.shift_right_logical(
                        d_raw, 4)
                    if per_core_head:
                        sv[j, 0, sl] = sv[j, 0, sl] + off16
                        dvo[j, 0, sl] = d_raw + off16
                    else:
                        dvo[j, 0, sl] = d_raw

        if per_core_head:
            w_id, nw = sid, 16
        else:
            w_id, nw = sid * 2 + cid, 32
        r0 = (_R * w_id) // nw
        r1 = (_R * (w_id + 1)) // nw
        nss = (r1 - r0) // _SSE

        def compute16(base, dst16):
            idx16 = _full16(base) + _iota16()
            as16 = plsc.load_gather(auxs, [idx16, _full16(0)])
            ad16 = plsc.load_gather(auxd, [idx16, _full16(1)])
            logit = as16 + ad16
            w16 = jnp.exp(jnp.where(logit >= 0, logit, 0.2 * logit))
            md16 = lax.bitwise_and(dst16, _full16(15))
            plsc.store_scatter(outw, [idx16, md16], w16)
            for kk in range(16):
                ck = plsc.load_gather(rows, [idx16, _full16(kk)])
                plsc.store_scatter(outb, [idx16, _full16(kk)], ck * w16)

        def do_block(nj):
            prep_ids(nj)
            hs = [pltpu.async_copy(t_hbm.at[sv.at[j, 0]],
                                   rows.at[pl.ds(j * 128, 128)], sem)
                  for j in range(nj)]
            hs += [pltpu.async_copy(aux_hbm.at[sv.at[j, 0]],
                                    auxs.at[pl.ds(j * 128, 128)], sem)
                   for j in range(nj)]
            hs += [pltpu.async_copy(aux_hbm.at[dvo.at[j, 0]],
                                    auxd.at[pl.ds(j * 128, 128)], sem)
                   for j in range(nj)]
            for h in hs:
                h.wait()
            for j in range(nj):
                for g in range(8):
                    compute16(j * 128 + g * 16, dv[j, 0, pl.ds(g * 16, 16)])
            hs = [pltpu.async_copy(outb.at[pl.ds(j * 128, 128)],
                                   acc.at[dv.at[j, 0]], sem, add=True)
                  for j in range(nj)]
            hs += [pltpu.async_copy(outw.at[pl.ds(j * 128, 128)],
                                    acc.at[dvw.at[j, 0]], sem, add=True)
                   for j in range(nj)]
            for h in hs:
                h.wait()
            for j in range(nj):
                for g in range(8):
                    base = j * 128 + g * 16
                    idx16 = _full16(base) + _iota16()
                    dst16 = dv[j, 0, pl.ds(g * 16, 16)]
                    md16 = lax.bitwise_and(dst16, _full16(15))
                    plsc.store_scatter(outw, [idx16, md16], z16)

        @pl.loop(0, nss)
        def _(t):
            rb = r0 + t * _SSE
            pltpu.sync_copy(srcb_hbm.at[pl.ds(rb, _SSE)], sv)
            pltpu.sync_copy(dstb_hbm.at[pl.ds(rb, _SSE)], dv)
            do_block(_SSE)

        @pl.loop(r0 + nss * _SSE, r1)
        def _(r):
            pltpu.sync_copy(srcb_hbm.at[pl.ds(r, 1)], sv.at[pl.ds(0, 1)])
            pltpu.sync_copy(dstb_hbm.at[pl.ds(r, 1)], dv.at[pl.ds(0, 1)])
            do_block(1)

        plsc.subcore_barrier()
        pltpu.sync_copy(acc.at[pl.ds(a0, a1 - a0)],
                        u_hbm.at[pl.ds(cid * _NA + a0, a1 - a0)])

    return k(Tg, aux, srcb, dstb)


def _d2_body(uh_ref, den_ref, b1_ref, w2_ref, as2_ref, ad2_ref,
             t2_ref, aux_ref):
    uh = uh_ref[...]
    den = den_ref[...]
    o0 = uh[0] / (den[0] + 1e-16)
    o1 = uh[1] / (den[1] + 1e-16)
    cat = jnp.concatenate([o0, o1], axis=1) + b1_ref[...]
    h1p = jnp.where(cat > 0, cat, jnp.exp(cat) - 1.0)
    h2t = jnp.dot(h1p.astype(jnp.bfloat16), w2_ref[...].astype(jnp.bfloat16),
                  preferred_element_type=jnp.float32)
    as_col = jnp.sum(h2t * as2_ref[...], axis=1, keepdims=True)
    ad_col = jnp.sum(h2t * ad2_ref[...], axis=1, keepdims=True)
    t2_ref[...] = h2t
    aux_ref[...] = jnp.concatenate(
        [as_col, ad_col, jnp.zeros((_BN, 6), jnp.float32)], axis=1)


def _dense2(U1h, den1, b1, W2, att_src2, att_dst2):
    return pl.pallas_call(
        _d2_body,
        grid=(_N // _BN,),
        in_specs=[
            pl.BlockSpec((2, _BN, 16), lambda i: (0, i, 0)),
            pl.BlockSpec((2, _BN, 1), lambda i: (0, i, 0)),
            pl.BlockSpec((1, 32), lambda i: (0, 0)),
            pl.BlockSpec((32, 16), lambda i: (0, 0)),
            pl.BlockSpec((1, 16), lambda i: (0, 0)),
            pl.BlockSpec((1, 16), lambda i: (0, 0)),
        ],
        out_specs=[
            pl.BlockSpec((_BN, 16), lambda i: (i, 0)),
            pl.BlockSpec((_BN, 8), lambda i: (i, 0)),
        ],
        out_shape=[
            jax.ShapeDtypeStruct((_N, 16), jnp.float32),
            jax.ShapeDtypeStruct((_N, 8), jnp.float32),
        ],
    )(U1h, den1, b1.reshape(1, 32), W2, att_src2, att_dst2)


def _d3_body(uh_ref, den_ref, b2_ref, out_ref):
    u = uh_ref[...][0] + uh_ref[...][1]
    dd = den_ref[...][0] + den_ref[...][1]
    out_ref[...] = u / (dd + 1e-16) + b2_ref[...]


def _dense3(U2h, den2, b2):
    return pl.pallas_call(
        _d3_body,
        grid=(_N // _BN,),
        in_specs=[
            pl.BlockSpec((2, _BN, 16), lambda i: (0, i, 0)),
            pl.BlockSpec((2, _BN, 1), lambda i: (0, i, 0)),
            pl.BlockSpec((1, 16), lambda i: (0, 0)),
        ],
        out_specs=pl.BlockSpec((_BN, 16), lambda i: (i, 0)),
        out_shape=jax.ShapeDtypeStruct((_N, 16), jnp.float32),
    )(U2h, den2, b2.reshape(1, 16))


def _p3_gather(h2f, srcb, dstb):
    mesh = plsc.VectorSubcoreMesh(core_axis_name="c", subcore_axis_name="s")

    @functools.partial(
        pl.kernel,
        out_type=jax.ShapeDtypeStruct((2, _E // 8, 128), jnp.float32),
        mesh=mesh,
        compiler_params=_SC_PARAMS,
        scratch_types=[
            pltpu.VMEM((_SS, 1, 128), jnp.int32),
            pltpu.VMEM((_SS, 1, 128), jnp.int32),
            pltpu.VMEM((_SS * 128, 16), jnp.float32),
            pltpu.VMEM((_SS * 128, 16), jnp.float32),
            pltpu.VMEM((_SS * 16, 128), jnp.float32),
            pltpu.VMEM((_SS * 16, 128), jnp.float32),
            pltpu.SemaphoreType.DMA,
        ],
    )
    def k(h2f_hbm, srcb_hbm, dstb_hbm, ef_hbm, sv, dv, srows, drows,
          spk, dpk, sem):
        def repack(nrow):
            for r in range(nrow):
                for m in range(8):
                    spk[r, pl.ds(m * 16, 16)] = srows[8 * r + m, pl.ds(0, 16)]
                    dpk[r, pl.ds(m * 16, 16)] = drows[8 * r + m, pl.ds(0, 16)]
        cid = lax.axis_index("c")
        sid = lax.axis_index("s")
        w = sid * 2 + cid
        r0 = (_R * w) // 32
        r1 = (_R * (w + 1)) // 32
        nss = (r1 - r0) // _SS

        @pl.loop(0, nss)
        def _(t):
            rb = r0 + t * _SS
            pltpu.sync_copy(srcb_hbm.at[pl.ds(rb, _SS)], sv)
            pltpu.sync_copy(dstb_hbm.at[pl.ds(rb, _SS)], dv)
            hs = []
            for j in range(_SS):
                hs.append(pltpu.async_copy(
                    h2f_hbm.at[sv.at[j, 0]], srows.at[pl.ds(j * 128, 128)], sem))
                hs.append(pltpu.async_copy(
                    h2f_hbm.at[dv.at[j, 0]], drows.at[pl.ds(j * 128, 128)], sem))
            for h in hs:
                h.wait()
            repack(_SS * 16)
            pltpu.sync_copy(spk, ef_hbm.at[0].at[pl.ds(rb * 16, _SS * 16)])
            pltpu.sync_copy(dpk, ef_hbm.at[1].at[pl.ds(rb * 16, _SS * 16)])

        @pl.loop(r0 + nss * _SS, r1)
        def _(r):
            pltpu.sync_copy(srcb_hbm.at[pl.ds(r, 1)], sv.at[pl.ds(0, 1)])
            pltpu.sync_copy(dstb_hbm.at[pl.ds(r, 1)], dv.at[pl.ds(0, 1)])
            pltpu.async_copy(
                h2f_hbm.at[sv.at[0, 0]], srows.at[pl.ds(0, 128)], sem).wait()
            pltpu.async_copy(
                h2f_hbm.at[dv.at[0, 0]], drows.at[pl.ds(0, 128)], sem).wait()
            repack(16)
            pltpu.sync_copy(spk.at[pl.ds(0, 16)],
                            ef_hbm.at[0].at[pl.ds(r * 16, 16)])
            pltpu.sync_copy(dpk.at[pl.ds(0, 16)],
                            ef_hbm.at[1].at[pl.ds(r * 16, 16)])

    return k(h2f, srcb, dstb)


def _d4_body(ef0_ref, ef1_ref, bds_ref, bdd_ref, ba8_ref, bdw_ref, bb_ref,
             out_ref):
    es = ef0_ref[...][0].astype(jnp.bfloat16)
    ed = ef1_ref[...][0].astype(jnp.bfloat16)
    hmid = (jnp.dot(es, bds_ref[...].astype(jnp.bfloat16),
                    preferred_element_type=jnp.float32)
            + jnp.dot(ed, bdd_ref[...].astype(jnp.bfloat16),
                      preferred_element_type=jnp.float32)
            + ba8_ref[...])
    hmid = jnp.maximum(hmid, 0.0)
    fl = jnp.dot(hmid.astype(jnp.bfloat16), bdw_ref[...].astype(jnp.bfloat16),
                 preferred_element_type=jnp.float32) + bb_ref[...]
    out_ref[...] = jnp.maximum(fl, 0.0)


def _dense4(ef, Wa, ba, Wb, bb):
    eye8 = jnp.eye(8, dtype=jnp.float32)
    bds = jnp.kron(eye8, Wa[:16])
    bdd = jnp.kron(eye8, Wa[16:])
    bdw = jnp.kron(eye8, Wb)
    ba8 = jnp.tile(ba, 8).reshape(1, 128)
    b8 = _BE // 8
    return pl.pallas_call(
        _d4_body,
        grid=(_E // _BE,),
        in_specs=[
            pl.BlockSpec((1, b8, 128), lambda i: (0, i, 0)),
            pl.BlockSpec((1, b8, 128), lambda i: (1, i, 0)),
            pl.BlockSpec((128, 128), lambda i: (0, 0)),
            pl.BlockSpec((128, 128), lambda i: (0, 0)),
            pl.BlockSpec((1, 128), lambda i: (0, 0)),
            pl.BlockSpec((128, 8), lambda i: (0, 0)),
            pl.BlockSpec((1, 1), lambda i: (0, 0)),
        ],
        out_specs=pl.BlockSpec((b8, 8), lambda i: (i, 0)),
        out_shape=jax.ShapeDtypeStruct((_E // 8, 8), jnp.float32),
    )(ef, ef, bds, bdd, ba8, bdw, bb.reshape(1, 1))


def kernel(x, edge_index, W1, att_src1, att_dst1, b1, W2, att_src2, att_dst2,
           b2, Wa, ba, Wb, bb):
    srcb = edge_index[0].reshape(_R, 1, 128)
    dstb = edge_index[1].reshape(_R, 1, 128)
    na = _N + _N // 16
    Tg1, aux1 = _dense1(x, W1, att_src1, att_dst1)
    U1 = _edge_pass(Tg1.reshape(2 * _N, 16), aux1.reshape(2 * _N, 8),
                    srcb, dstb, True).reshape(2, na, 16)
    U1h = U1[:, :_N, :]
    den1 = U1[:, _N:, :].reshape(2, _N, 1)
    T2, aux2 = _dense2(U1h, den1, b1, W2, att_src2, att_dst2)
    U2 = _edge_pass(T2, aux2, srcb, dstb, False).reshape(2, na, 16)
    U2h = U2[:, :_N, :]
    den2 = U2[:, _N:, :].reshape(2, _N, 1)
    h2f = _dense3(U2h, den2, b2)
    ef = _p3_gather(h2f, srcb, dstb)
    flows = _dense4(ef, Wa, ba, Wb, bb)
    return flows.reshape(_E)
